# baseline (speedup 1.0000x reference)
import functools

import jax
import jax.numpy as jnp
from jax import lax
from jax.experimental import pallas as pl
from jax.experimental.pallas import tpu as pltpu
from jax.experimental.pallas import tpu_sc as plsc

_HS = 768
_E = 64
_TOPK = 8
_BT = 4096
_N = 32768
_NW = 32
_ROWS = _N // _NW


def _score_body(x_ref, w_ref, p_ref):
    x = x_ref[...]
    w = w_ref[...]
    logits = jax.lax.dot_general(
        x, w, (((1,), (1,)), ((), ())),
        preferred_element_type=jnp.float32)
    m = jnp.max(logits, axis=1, keepdims=True)
    p = jnp.exp(logits - m)
    p_ref[...] = p / jnp.sum(p, axis=1, keepdims=True)


@jax.jit
def _scores(xf, W):
    return pl.pallas_call(
        _score_body,
        grid=(_N // _BT,),
        in_specs=[
            pl.BlockSpec((_BT, _HS), lambda i: (i, 0)),
            pl.BlockSpec((_E, _HS), lambda i: (0, 0)),
        ],
        out_specs=pl.BlockSpec((_BT, _E), lambda i: (i, 0)),
        out_shape=jax.ShapeDtypeStruct((_N, _E), jnp.float32),
    )(xf, W)


def _merge_desc(ka, ia, kb, ib):
    krb = lax.rev(kb, (0,))
    irb = lax.rev(ib, (0,))
    take_a = ka >= krb
    mk = jnp.where(take_a, ka, krb)
    mi = jnp.where(take_a, ia, irb)
    return plsc.sort_key_val(mk, mi, descending=True)


def _topk_body(p_hbm, out_hbm, chunk, outc):
    wid = lax.axis_index("s") * 2 + lax.axis_index("c")
    base = wid * _ROWS
    pltpu.sync_copy(p_hbm.at[pl.ds(base * _E, _ROWS * _E)], chunk)
    lane = lax.broadcasted_iota(jnp.int32, (16,), 0)

    def body(r, carry):
        ks = []
        vs = []
        for j in range(4):
            kj = chunk[pl.ds(r * _E + 16 * j, 16)]
            ij = lane + 16 * j
            skj, sij = plsc.sort_key_val(kj, ij, descending=True)
            ks.append(skj)
            vs.append(sij)
        k01, i01 = _merge_desc(ks[0], vs[0], ks[1], vs[1])
        k23, i23 = _merge_desc(ks[2], vs[2], ks[3], vs[3])
        kf, if_ = _merge_desc(k01, i01, k23, i23)
        comb = jnp.where(lane < 8, kf, plsc.bitcast(lax.rev(if_, (0,)),
                                                    jnp.float32))
        outc[pl.ds(r * 16, 16)] = comb
        return carry

    lax.fori_loop(0, _ROWS, body, 0)
    pltpu.sync_copy(outc, out_hbm.at[pl.ds(base * 16, _ROWS * 16)])


@jax.jit
def _topk_sc(pflat):
    mesh = plsc.VectorSubcoreMesh(core_axis_name="c", subcore_axis_name="s")
    fn = functools.partial(
        pl.kernel,
        mesh=mesh,
        compiler_params=pltpu.CompilerParams(needs_layout_passes=False),
        out_type=jax.ShapeDtypeStruct((_N * 16,), jnp.float32),
        scratch_types=[
            pltpu.VMEM((_ROWS * _E,), jnp.float32),
            pltpu.VMEM((_ROWS * 16,), jnp.float32),
        ],
    )(_topk_body)
    return fn(pflat)


def kernel(x, W):
    xf = x.reshape(-1, x.shape[-1])
    p = _scores(xf, W)
    out = _topk_sc(p.reshape(-1)).reshape(_N, 16)
    w8 = out[:, :_TOPK]
    i8 = jax.lax.bitcast_convert_type(out[:, 16 - _TOPK:], jnp.int32)[:, ::-1]
    return (w8, i8)


# speedup vs baseline: 2.5838x; 2.5838x over previous
import jax
import jax.numpy as jnp
from jax.experimental import pallas as pl

_HS = 768
_E = 64
_TOPK = 8
_BT = 4096


def _router_body(x_ref, w_ref, out_ref):
    x = x_ref[...]
    w = w_ref[...]
    logits = jax.lax.dot_general(
        w, x, (((1,), (1,)), ((), ())),
        preferred_element_type=jnp.float32)
    m = jnp.max(logits, axis=0, keepdims=True)
    p = jnp.exp(logits - m)
    rdenom = 1.0 / jnp.sum(p, axis=0, keepdims=True)

    eidx = jax.lax.broadcasted_iota(jnp.int32, (_E, _BT), 0)
    vals = p
    for k in range(_TOPK):
        mk = jnp.max(vals, axis=0, keepdims=True)
        hit = vals == mk
        idx = jnp.min(jnp.where(hit, eidx, _E), axis=0, keepdims=True)
        out_ref[pl.ds(k, 1), :] = mk * rdenom
        out_ref[pl.ds(_TOPK + k, 1), :] = jax.lax.bitcast_convert_type(
            idx, jnp.float32)
        vals = jnp.where(eidx == idx, -1.0, vals)


@jax.jit
def _router(xf, W):
    n = xf.shape[0]
    grid = (n // _BT,)
    return pl.pallas_call(
        _router_body,
        grid=grid,
        in_specs=[
            pl.BlockSpec((_BT, _HS), lambda i: (i, 0)),
            pl.BlockSpec((_E, _HS), lambda i: (0, 0)),
        ],
        out_specs=pl.BlockSpec((2 * _TOPK, _BT), lambda i: (0, i)),
        out_shape=jax.ShapeDtypeStruct((2 * _TOPK, n), jnp.float32),
    )(xf, W)


def kernel(x, W):
    xf = x.reshape(-1, x.shape[-1])
    out = _router(xf, W).T
    w8 = out[:, :_TOPK]
    i8 = jax.lax.bitcast_convert_type(out[:, _TOPK:], jnp.int32)
    return (w8, i8)


# speedup vs baseline: 2.6453x; 1.0238x over previous
import functools

import jax
import jax.numpy as jnp
from jax.experimental import pallas as pl
from jax.experimental.pallas import tpu as pltpu

_HS = 768
_E = 64
_TOPK = 8
_BT = 8192


def _router_body(x_ref, w_ref, wout_ref, iout_ref):
    x = x_ref[...]
    w = w_ref[...]
    logits = jax.lax.dot_general(
        w, x, (((1,), (1,)), ((), ())),
        preferred_element_type=jnp.float32)
    m = jnp.max(logits, axis=0, keepdims=True)
    p = jnp.exp(logits - m)
    rdenom = 1.0 / jnp.sum(p, axis=0, keepdims=True)

    eidx = jax.lax.broadcasted_iota(jnp.int32, (_E, _BT), 0)
    vals = p
    for k in range(_TOPK):
        mk = jnp.max(vals, axis=0, keepdims=True)
        hit = vals == mk
        idx = jnp.min(jnp.where(hit, eidx, _E), axis=0, keepdims=True)
        wout_ref[pl.ds(k, 1), :] = mk * rdenom
        iout_ref[pl.ds(k, 1), :] = idx
        vals = jnp.where(eidx == idx, -1.0, vals)


@jax.jit
def _router(xf, W):
    n = xf.shape[0]
    grid = (n // _BT,)
    return pl.pallas_call(
        _router_body,
        grid=grid,
        in_specs=[
            pl.BlockSpec((_BT, _HS), lambda i: (i, 0)),
            pl.BlockSpec((_E, _HS), lambda i: (0, 0)),
        ],
        out_specs=[
            pl.BlockSpec((_TOPK, _BT), lambda i: (0, i)),
            pl.BlockSpec((_TOPK, _BT), lambda i: (0, i)),
        ],
        out_shape=[
            jax.ShapeDtypeStruct((_TOPK, n), jnp.float32),
            jax.ShapeDtypeStruct((_TOPK, n), jnp.int32),
        ],
    )(xf, W)


def kernel(x, W):
    xf = x.reshape(-1, x.shape[-1])
    wT, iT = _router(xf, W)
    return (wT.T, iT.T)


# speedup vs baseline: 2.6869x; 1.0157x over previous
import functools

import jax
import jax.numpy as jnp
from jax.experimental import pallas as pl
from jax.experimental.pallas import tpu as pltpu

_HS = 768
_E = 64
_TOPK = 8
_BT = 4096


def _router_body(x_ref, w_ref, wout_ref, iout_ref):
    x = x_ref[...]
    w = w_ref[...]
    logits = jax.lax.dot_general(
        w, x, (((1,), (1,)), ((), ())),
        preferred_element_type=jnp.float32)
    m = jnp.max(logits, axis=0, keepdims=True)
    p = jnp.exp(logits - m)
    rdenom = 1.0 / jnp.sum(p, axis=0, keepdims=True)

    eidx = jax.lax.broadcasted_iota(jnp.int32, (_E, _BT), 0)
    vals = p
    for k in range(_TOPK):
        mk = jnp.max(vals, axis=0, keepdims=True)
        hit = vals == mk
        idx = jnp.min(jnp.where(hit, eidx, _E), axis=0, keepdims=True)
        wout_ref[pl.ds(k, 1), :] = mk * rdenom
        iout_ref[pl.ds(k, 1), :] = idx
        vals = jnp.where(eidx == idx, -1.0, vals)


@jax.jit
def _router(xf, W):
    n = xf.shape[0]
    grid = (n // _BT,)
    return pl.pallas_call(
        _router_body,
        grid=grid,
        in_specs=[
            pl.BlockSpec((_BT, _HS), lambda i: (i, 0)),
            pl.BlockSpec((_E, _HS), lambda i: (0, 0)),
        ],
        out_specs=[
            pl.BlockSpec((_TOPK, _BT), lambda i: (0, i)),
            pl.BlockSpec((_TOPK, _BT), lambda i: (0, i)),
        ],
        out_shape=[
            jax.ShapeDtypeStruct((_TOPK, n), jnp.float32),
            jax.ShapeDtypeStruct((_TOPK, n), jnp.int32),
        ],
    )(xf, W)


def kernel(x, W):
    xf = x.reshape(-1, x.shape[-1])
    wT, iT = _router(xf, W)
    return (wT.T, iT.T)


# speedup vs baseline: 2.6924x; 1.0021x over previous
import jax
import jax.numpy as jnp
from jax.experimental import pallas as pl

_HS = 768
_E = 64
_TOPK = 8
_BT = 4096


def _router_body(x_ref, w_ref, wout_ref, iout_ref):
    x = x_ref[...]
    w = w_ref[...]
    logits = jax.lax.dot_general(
        w, x, (((1,), (1,)), ((), ())),
        preferred_element_type=jnp.float32)
    m = jnp.max(logits, axis=0, keepdims=True)
    p = jnp.exp(logits - m)
    rdenom = 1.0 / jnp.sum(p, axis=0, keepdims=True)

    eidx = jax.lax.broadcasted_iota(jnp.int32, (_E, _BT), 0)
    vals = p
    for k in range(_TOPK):
        mk = jnp.max(vals, axis=0, keepdims=True)
        hit = vals == mk
        idx = jnp.min(jnp.where(hit, eidx, _E), axis=0, keepdims=True)
        wout_ref[pl.ds(k, 1), :] = mk * rdenom
        iout_ref[pl.ds(k, 1), :] = idx
        vals = jnp.where(eidx == idx, -1.0, vals)


@jax.jit
def _router(xf, W):
    n = xf.shape[0]
    grid = (n // _BT,)
    return pl.pallas_call(
        _router_body,
        grid=grid,
        in_specs=[
            pl.BlockSpec((_BT, _HS), lambda i: (i, 0)),
            pl.BlockSpec((_E, _HS), lambda i: (0, 0)),
        ],
        out_specs=[
            pl.BlockSpec((_TOPK, _BT), lambda i: (0, i)),
            pl.BlockSpec((_TOPK, _BT), lambda i: (0, i)),
        ],
        out_shape=[
            jax.ShapeDtypeStruct((_TOPK, n), jnp.float32),
            jax.ShapeDtypeStruct((_TOPK, n), jnp.int32),
        ],
    )(xf, W)


def kernel(x, W):
    xf = x.reshape(-1, x.shape[-1])
    wT, iT = _router(xf, W)
    return (wT.T, iT.T)
